# baseline (speedup 1.0000x reference)
import functools

import jax
import jax.numpy as jnp
from jax import lax
from jax.experimental import pallas as pl
from jax.experimental.pallas import tpu as pltpu
from jax.experimental.pallas import tpu_sc as plsc

N_USERS = 50000
N_NODES = 100000
H = 16
N_LAYERS = 3
N_EDGES = 1600000
NS = 16
E = 128
SUB = 25
NE_PAD = 1638400
PER_TILE = NE_PAD // NS
SUPERS = PER_TILE // (SUB * E)
NB = 5
NPT = N_NODES // NS
WB = 125
ZCH = NPT // WB

_mesh = plsc.VectorSubcoreMesh(core_axis_name="c", subcore_axis_name="s")


def _body(inp, rows, cols, vals, out, tables, acc,
          rows2d, cols2d, vals2d, msg, zbuf, buf, buf2,
          sem0, sem1, sem2, sem3, sem4):
    sems = (sem0, sem1, sem2, sem3, sem4)
    c = lax.axis_index("c")
    s = lax.axis_index("s")
    m_base = s * (PER_TILE // E)
    n0 = s * NPT

    def _zero(r, _):
        zbuf[r, :] = jnp.zeros((H,), jnp.float32)
        return 0
    lax.fori_loop(0, WB, _zero, 0)

    for l in (1, 2, 3):
        def _zacc(k, _):
            pltpu.sync_copy(zbuf, acc.at[pl.ds(n0 + k * WB, WB)])
            return 0
        lax.fori_loop(0, ZCH, _zacc, 0)
        plsc.subcore_barrier()

        src = inp if l == 1 else tables
        base = c * N_NODES if l == 1 else (2 * (l - 2) + c) * N_NODES

        def _drain(j, b):
            pltpu.make_async_copy(
                src.at[cols2d.at[j]], msg.at[b], sems[b]).wait()

            def _scale(g, _):
                vv = vals2d[j, pl.ds(16 * g, 16)]
                for jj in range(16):
                    e = 16 * g + jj
                    msg[b, e, :] = msg[b, e, :] * vv[jj]
                return 0
            lax.fori_loop(0, E // 16, _scale, 0)
            pltpu.sync_copy(msg.at[b], acc.at[rows2d.at[j]], add=True)

        def _super(si, _):
            m0 = m_base + si * SUB
            pltpu.sync_copy(rows.at[pl.ds(m0, SUB)], rows2d)
            pltpu.sync_copy(cols.at[pl.ds(m0, SUB)], cols2d)
            pltpu.sync_copy(vals.at[pl.ds(m0, SUB)], vals2d)

            def _off(j, _):
                for g in range(E // 16):
                    cols2d[j, pl.ds(16 * g, 16)] = (
                        cols2d[j, pl.ds(16 * g, 16)] + base)
                return 0
            lax.fori_loop(0, SUB, _off, 0)

            for b in range(NB):
                pltpu.async_copy(src.at[cols2d.at[b]], msg.at[b], sems[b])

            def _steady(i, _):
                j0 = i * NB
                for b in range(NB):
                    _drain(j0 + b, b)
                    pltpu.async_copy(
                        src.at[cols2d.at[j0 + b + NB]], msg.at[b], sems[b])
                return 0
            lax.fori_loop(0, SUB // NB - 1, _steady, 0)
            for b in range(NB):
                _drain(SUB - NB + b, b)
            return 0
        lax.fori_loop(0, SUPERS, _super, 0)
        plsc.subcore_barrier()

        tb = (2 * (l - 1) + c) * N_NODES

        def _wb(k, _):
            pltpu.sync_copy(acc.at[pl.ds(n0 + k * WB, WB)], buf)
            pltpu.sync_copy(buf, tables.at[pl.ds(tb + n0 + k * WB, WB)])
            return 0
        lax.fori_loop(0, ZCH, _wb, 0)
        plsc.subcore_barrier()

    def _mean(k, _):
        noff = n0 + k * WB
        pltpu.sync_copy(inp.at[pl.ds(c * N_NODES + noff, WB)], buf)
        for l in (1, 2, 3):
            pltpu.sync_copy(
                tables.at[pl.ds((2 * (l - 1) + c) * N_NODES + noff, WB)], buf2)
            if l < 3:
                def _add(r, _):
                    buf[r, :] = buf[r, :] + buf2[r, :]
                    return 0
            else:
                def _add(r, _):
                    buf[r, :] = (buf[r, :] + buf2[r, :]) * 0.25
                    return 0
            lax.fori_loop(0, WB, _add, 0)
        pltpu.sync_copy(buf, out.at[pl.ds(c * N_NODES + noff, WB)])
        return 0
    lax.fori_loop(0, ZCH, _mean, 0)


_gcn = functools.partial(
    pl.kernel,
    mesh=_mesh,
    compiler_params=pltpu.CompilerParams(use_tc_tiling_on_sc=False),
    out_type=(
        jax.ShapeDtypeStruct((2 * N_NODES, H), jnp.float32),
        jax.ShapeDtypeStruct((2 * N_LAYERS * N_NODES, H), jnp.float32),
    ),
    scratch_types=[
        pltpu.VMEM_SHARED((N_NODES, H), jnp.float32),
        pltpu.VMEM((SUB, E), jnp.int32),
        pltpu.VMEM((SUB, E), jnp.int32),
        pltpu.VMEM((SUB, E), jnp.float32),
        pltpu.VMEM((NB, E, H), jnp.float32),
        pltpu.VMEM((WB, H), jnp.float32),
        pltpu.VMEM((WB, H), jnp.float32),
        pltpu.VMEM((WB, H), jnp.float32),
    ] + [pltpu.SemaphoreType.DMA] * NB,
)(_body)


def kernel(edge_index, edge_values, emb_user, emb_item):
    all_emb = jnp.concatenate([emb_user, emb_item], axis=0)
    inp = jnp.concatenate([all_emb[:, :H], all_emb[:, H:]], axis=0)
    pad = NE_PAD - N_EDGES
    zi = jnp.zeros((pad,), jnp.int32)
    rows2d = jnp.concatenate([edge_index[0], zi]).reshape(NE_PAD // E, E)
    cols2d = jnp.concatenate([edge_index[1], zi]).reshape(NE_PAD // E, E)
    vals2d = jnp.concatenate(
        [edge_values, jnp.zeros((pad,), jnp.float32)]).reshape(NE_PAD // E, E)
    out, _ = _gcn(inp, rows2d, cols2d, vals2d)
    full = jnp.concatenate([out[:N_NODES], out[N_NODES:]], axis=1)
    return full[:N_USERS], full[N_USERS:]


# speedup vs baseline: 1.4545x; 1.4545x over previous
import functools

import jax
import jax.numpy as jnp
from jax import lax
from jax.experimental import pallas as pl
from jax.experimental.pallas import tpu as pltpu
from jax.experimental.pallas import tpu_sc as plsc

N_USERS = 50000
N_NODES = 100000
H = 16
N_LAYERS = 3
N_EDGES = 1600000
NS = 16
E = 80
SUB = 25
PER_TILE = N_EDGES // NS
SUPERS = PER_TILE // (SUB * E)
NB = 5
NPT = N_NODES // NS
WB = 250
ZCH = NPT // WB

_mesh = plsc.VectorSubcoreMesh(core_axis_name="c", subcore_axis_name="s")


def _body(inp, rows, cols, vals, out, tables, acc,
          rows3d, cols3d, vals3d, msg, zbuf, buf, buf2,
          sem0, sem1, sem2, sem3, sem4, semm):
    sems = (sem0, sem1, sem2, sem3, sem4)
    c = lax.axis_index("c")
    s = lax.axis_index("s")
    m_base = s * (PER_TILE // E)
    n0 = s * NPT

    def _zero(r, _):
        zbuf[r, :] = jnp.zeros((H,), jnp.float32)
        return 0
    lax.fori_loop(0, WB, _zero, 0)

    def _meta_fire(si):
        m0 = m_base + si * SUB
        p = lax.rem(si, 2)
        pltpu.async_copy(rows.at[pl.ds(m0, SUB)], rows3d.at[p], semm)
        pltpu.async_copy(cols.at[pl.ds(m0, SUB)], cols3d.at[p], semm)
        pltpu.async_copy(vals.at[pl.ds(m0, SUB)], vals3d.at[p], semm)

    def _meta_wait(si):
        m0 = m_base + si * SUB
        p = lax.rem(si, 2)
        pltpu.make_async_copy(rows.at[pl.ds(m0, SUB)], rows3d.at[p],
                              semm).wait()
        pltpu.make_async_copy(cols.at[pl.ds(m0, SUB)], cols3d.at[p],
                              semm).wait()
        pltpu.make_async_copy(vals.at[pl.ds(m0, SUB)], vals3d.at[p],
                              semm).wait()

    for l in (1, 2, 3):
        _meta_fire(0)

        def _zacc(k, _):
            pltpu.sync_copy(zbuf, acc.at[pl.ds(n0 + k * WB, WB)])
            return 0
        lax.fori_loop(0, ZCH, _zacc, 0)
        plsc.subcore_barrier()

        src = inp if l == 1 else tables
        base = c * N_NODES if l == 1 else (2 * (l - 2) + c) * N_NODES

        def _drain(p, j, b):
            pltpu.make_async_copy(
                src.at[cols3d.at[p, j]], msg.at[b], sems[b]).wait()

            def _scale(g, _):
                vv = vals3d[p, j, pl.ds(16 * g, 16)]
                for jj in range(16):
                    e = 16 * g + jj
                    msg[b, e, :] = msg[b, e, :] * vv[jj]
                return 0
            lax.fori_loop(0, E // 16, _scale, 0)
            pltpu.sync_copy(msg.at[b], acc.at[rows3d.at[p, j]], add=True)

        def _super(si, _):
            p = lax.rem(si, 2)
            _meta_wait(si)

            @pl.when(si + 1 < SUPERS)
            def _():
                _meta_fire(si + 1)

            def _off(j, _):
                for g in range(E // 16):
                    cols3d[p, j, pl.ds(16 * g, 16)] = (
                        cols3d[p, j, pl.ds(16 * g, 16)] + base)
                return 0
            lax.fori_loop(0, SUB, _off, 0)

            for b in range(NB):
                pltpu.async_copy(src.at[cols3d.at[p, b]], msg.at[b], sems[b])

            def _steady(i, _):
                j0 = i * NB
                for b in range(NB):
                    _drain(p, j0 + b, b)
                    pltpu.async_copy(
                        src.at[cols3d.at[p, j0 + b + NB]], msg.at[b], sems[b])
                return 0
            lax.fori_loop(0, SUB // NB - 1, _steady, 0)
            for b in range(NB):
                _drain(p, SUB - NB + b, b)
            return 0
        lax.fori_loop(0, SUPERS, _super, 0)
        plsc.subcore_barrier()

        if l < 3:
            tb = (2 * (l - 1) + c) * N_NODES

            def _wb(k, _):
                pltpu.sync_copy(acc.at[pl.ds(n0 + k * WB, WB)], buf)
                pltpu.sync_copy(buf, tables.at[pl.ds(tb + n0 + k * WB, WB)])
                return 0
            lax.fori_loop(0, ZCH, _wb, 0)
            plsc.subcore_barrier()

    def _mean(k, _):
        noff = n0 + k * WB
        pltpu.sync_copy(inp.at[pl.ds(c * N_NODES + noff, WB)], buf)
        for l in (1, 2, 3):
            if l < 3:
                pltpu.sync_copy(
                    tables.at[pl.ds((2 * (l - 1) + c) * N_NODES + noff, WB)],
                    buf2)
            else:
                pltpu.sync_copy(acc.at[pl.ds(noff, WB)], buf2)
            if l < 3:
                def _add(r, _):
                    buf[r, :] = buf[r, :] + buf2[r, :]
                    return 0
            else:
                def _add(r, _):
                    buf[r, :] = (buf[r, :] + buf2[r, :]) * 0.25
                    return 0
            lax.fori_loop(0, WB, _add, 0)
        pltpu.sync_copy(buf, out.at[pl.ds(c * N_NODES + noff, WB)])
        return 0
    lax.fori_loop(0, ZCH, _mean, 0)


_gcn = functools.partial(
    pl.kernel,
    mesh=_mesh,
    compiler_params=pltpu.CompilerParams(use_tc_tiling_on_sc=False),
    out_type=(
        jax.ShapeDtypeStruct((2 * N_NODES, H), jnp.float32),
        jax.ShapeDtypeStruct((2 * (N_LAYERS - 1) * N_NODES, H), jnp.float32),
    ),
    scratch_types=[
        pltpu.VMEM_SHARED((N_NODES, H), jnp.float32),
        pltpu.VMEM((2, SUB, E), jnp.int32),
        pltpu.VMEM((2, SUB, E), jnp.int32),
        pltpu.VMEM((2, SUB, E), jnp.float32),
        pltpu.VMEM((NB, E, H), jnp.float32),
        pltpu.VMEM((WB, H), jnp.float32),
        pltpu.VMEM((WB, H), jnp.float32),
        pltpu.VMEM((WB, H), jnp.float32),
    ] + [pltpu.SemaphoreType.DMA] * (NB + 1),
)(_body)


def kernel(edge_index, edge_values, emb_user, emb_item):
    all_emb = jnp.concatenate([emb_user, emb_item], axis=0)
    inp = jnp.concatenate([all_emb[:, :H], all_emb[:, H:]], axis=0)
    rows2d = edge_index[0].reshape(N_EDGES // E, E)
    cols2d = edge_index[1].reshape(N_EDGES // E, E)
    vals2d = edge_values.reshape(N_EDGES // E, E)
    out, _ = _gcn(inp, rows2d, cols2d, vals2d)
    full = jnp.concatenate([out[:N_NODES], out[N_NODES:]], axis=1)
    return full[:N_USERS], full[N_USERS:]


# speedup vs baseline: 1.6218x; 1.1150x over previous
import functools

import jax
import jax.numpy as jnp
from jax import lax
from jax.experimental import pallas as pl
from jax.experimental.pallas import tpu as pltpu
from jax.experimental.pallas import tpu_sc as plsc

N_USERS = 50000
N_NODES = 100000
H = 16
N_LAYERS = 3
N_EDGES = 1600000
NS = 16
E = 80
SUB = 25
PER_TILE = N_EDGES // NS
SUPERS = PER_TILE // (SUB * E)
NB = 5
NPT = N_NODES // NS
WB = 250
ZCH = NPT // WB

_mesh = plsc.VectorSubcoreMesh(core_axis_name="c", subcore_axis_name="s")


def _body(emb_u, emb_i, rows, cols, vals, out_u, out_i, tables, acc,
          rows3d, cols3d, vals3d, msg, zbuf, buf, buf2,
          sem0, sem1, sem2, sem3, sem4, semm):
    sems = (sem0, sem1, sem2, sem3, sem4)
    c = lax.axis_index("c")
    s = lax.axis_index("s")
    m_base = s * (PER_TILE // E)
    n0 = s * NPT

    def _zero(r, _):
        zbuf[r, :] = jnp.zeros((H,), jnp.float32)
        return 0
    lax.fori_loop(0, WB, _zero, 0)

    def _meta_fire(si):
        m0 = m_base + si * SUB
        p = lax.rem(si, 2)
        pltpu.async_copy(rows.at[pl.ds(m0, SUB)], rows3d.at[p], semm)
        pltpu.async_copy(cols.at[pl.ds(m0, SUB)], cols3d.at[p], semm)
        pltpu.async_copy(vals.at[pl.ds(m0, SUB)], vals3d.at[p], semm)

    def _meta_wait(si):
        m0 = m_base + si * SUB
        p = lax.rem(si, 2)
        pltpu.make_async_copy(rows.at[pl.ds(m0, SUB)], rows3d.at[p],
                              semm).wait()
        pltpu.make_async_copy(cols.at[pl.ds(m0, SUB)], cols3d.at[p],
                              semm).wait()
        pltpu.make_async_copy(vals.at[pl.ds(m0, SUB)], vals3d.at[p],
                              semm).wait()

    def _cp0(k, _):
        g0 = n0 + k * WB

        @pl.when(s < NS // 2)
        def _():
            pltpu.sync_copy(
                emb_u.at[pl.ds(g0, WB), pl.ds(c * H, H)], buf)

        @pl.when(s >= NS // 2)
        def _():
            pltpu.sync_copy(
                emb_i.at[pl.ds(g0 - N_USERS, WB), pl.ds(c * H, H)], buf)
        pltpu.sync_copy(buf, tables.at[pl.ds(c * N_NODES + g0, WB)])
        return 0
    lax.fori_loop(0, ZCH, _cp0, 0)
    plsc.subcore_barrier()

    for l in (1, 2, 3):
        _meta_fire(0)

        def _zacc(k, _):
            pltpu.sync_copy(zbuf, acc.at[pl.ds(n0 + k * WB, WB)])
            return 0
        lax.fori_loop(0, ZCH, _zacc, 0)
        plsc.subcore_barrier()

        src = tables
        base = (2 * (l - 1) + c) * N_NODES

        def _drain(p, j, b):
            pltpu.make_async_copy(
                src.at[cols3d.at[p, j]], msg.at[b], sems[b]).wait()

            def _scale(g, _):
                vv = vals3d[p, j, pl.ds(16 * g, 16)]
                for jj in range(16):
                    e = 16 * g + jj
                    msg[b, e, :] = msg[b, e, :] * vv[jj]
                return 0
            lax.fori_loop(0, E // 16, _scale, 0)
            pltpu.sync_copy(msg.at[b], acc.at[rows3d.at[p, j]], add=True)

        def _super(si, _):
            p = lax.rem(si, 2)
            _meta_wait(si)

            @pl.when(si + 1 < SUPERS)
            def _():
                _meta_fire(si + 1)

            def _off(j, _):
                for g in range(E // 16):
                    cols3d[p, j, pl.ds(16 * g, 16)] = (
                        cols3d[p, j, pl.ds(16 * g, 16)] + base)
                return 0
            lax.fori_loop(0, SUB, _off, 0)

            for b in range(NB):
                pltpu.async_copy(src.at[cols3d.at[p, b]], msg.at[b], sems[b])

            def _steady(i, _):
                j0 = i * NB
                for b in range(NB):
                    _drain(p, j0 + b, b)
                    pltpu.async_copy(
                        src.at[cols3d.at[p, j0 + b + NB]], msg.at[b], sems[b])
                return 0
            lax.fori_loop(0, SUB // NB - 1, _steady, 0)
            for b in range(NB):
                _drain(p, SUB - NB + b, b)
            return 0
        lax.fori_loop(0, SUPERS, _super, 0)
        plsc.subcore_barrier()

        if l < 3:
            tb = (2 * l + c) * N_NODES

            def _wb(k, _):
                pltpu.sync_copy(acc.at[pl.ds(n0 + k * WB, WB)], buf)
                pltpu.sync_copy(buf, tables.at[pl.ds(tb + n0 + k * WB, WB)])
                return 0
            lax.fori_loop(0, ZCH, _wb, 0)
            plsc.subcore_barrier()

    def _mean(k, _):
        noff = n0 + k * WB
        pltpu.sync_copy(tables.at[pl.ds(c * N_NODES + noff, WB)], buf)
        for l in (1, 2, 3):
            if l < 3:
                pltpu.sync_copy(
                    tables.at[pl.ds((2 * l + c) * N_NODES + noff, WB)],
                    buf2)
            else:
                pltpu.sync_copy(acc.at[pl.ds(noff, WB)], buf2)
            if l < 3:
                def _add(r, _):
                    buf[r, :] = buf[r, :] + buf2[r, :]
                    return 0
            else:
                def _add(r, _):
                    buf[r, :] = (buf[r, :] + buf2[r, :]) * 0.25
                    return 0
            lax.fori_loop(0, WB, _add, 0)

        @pl.when(s < NS // 2)
        def _():
            pltpu.sync_copy(
                buf, out_u.at[pl.ds(noff, WB), pl.ds(c * H, H)])

        @pl.when(s >= NS // 2)
        def _():
            pltpu.sync_copy(
                buf, out_i.at[pl.ds(noff - N_USERS, WB), pl.ds(c * H, H)])
        return 0
    lax.fori_loop(0, ZCH, _mean, 0)


_gcn = functools.partial(
    pl.kernel,
    mesh=_mesh,
    compiler_params=pltpu.CompilerParams(use_tc_tiling_on_sc=False),
    out_type=(
        jax.ShapeDtypeStruct((N_USERS, 2 * H), jnp.float32),
        jax.ShapeDtypeStruct((N_NODES - N_USERS, 2 * H), jnp.float32),
        jax.ShapeDtypeStruct((2 * N_LAYERS * N_NODES, H), jnp.float32),
    ),
    scratch_types=[
        pltpu.VMEM_SHARED((N_NODES, H), jnp.float32),
        pltpu.VMEM((2, SUB, E), jnp.int32),
        pltpu.VMEM((2, SUB, E), jnp.int32),
        pltpu.VMEM((2, SUB, E), jnp.float32),
        pltpu.VMEM((NB, E, H), jnp.float32),
        pltpu.VMEM((WB, H), jnp.float32),
        pltpu.VMEM((WB, H), jnp.float32),
        pltpu.VMEM((WB, H), jnp.float32),
    ] + [pltpu.SemaphoreType.DMA] * (NB + 1),
)(_body)


def kernel(edge_index, edge_values, emb_user, emb_item):
    rows2d = edge_index[0].reshape(N_EDGES // E, E)
    cols2d = edge_index[1].reshape(N_EDGES // E, E)
    vals2d = edge_values.reshape(N_EDGES // E, E)
    out_u, out_i, _ = _gcn(emb_user, emb_item, rows2d, cols2d, vals2d)
    return out_u, out_i
